# baseline (speedup 1.0000x reference)
import functools

import jax
import jax.numpy as jnp
from jax import lax
from jax.experimental import pallas as pl
from jax.experimental.pallas import tpu as pltpu
from jax.experimental.pallas import tpu_sc as plsc

N = 10000
E = 320000
D_IN = 128
D_HID = 128
D_OUT = 2
D_PAD = 16

NC = 2
NS = 16
NW = NC * NS
CHUNK = 64
NCHUNK = 158
EPT = CHUNK * NCHUNK
EPAD = NW * EPT
NROW = N + 8
ROWS_PT = 1000
N_IO_TILES = N // ROWS_PT

_MESH = plsc.VectorSubcoreMesh(
    core_axis_name="c", subcore_axis_name="s", num_cores=NC, num_subcores=NS
)
_SC_PARAMS = pltpu.CompilerParams(use_tc_tiling_on_sc=False)


def _wid():
    return lax.axis_index("c") * NS + lax.axis_index("s")


@functools.partial(
    pl.kernel,
    out_type=jax.ShapeDtypeStruct((NC, N, D_PAD), jnp.float32),
    mesh=_MESH,
    scratch_types=[
        pltpu.VMEM((NCHUNK, CHUNK), jnp.int32),
        pltpu.VMEM((CHUNK, D_PAD), jnp.float32),
        pltpu.VMEM_SHARED((NROW, D_PAD), jnp.float32),
    ],
    compiler_params=_SC_PARAMS,
)
def _sc_degree(dst3_hbm, ones_hbm, zeros_hbm, out_hbm, dst_v, ones_v, acc):
    c = lax.axis_index("c")
    s = lax.axis_index("s")
    wid = _wid()

    @pl.when(s < N_IO_TILES)
    def _zero():
        sl = pl.ds(s * ROWS_PT, ROWS_PT)
        pltpu.sync_copy(zeros_hbm.at[sl], acc.at[sl])

    pltpu.sync_copy(dst3_hbm.at[wid], dst_v)
    pltpu.sync_copy(ones_hbm, ones_v)
    plsc.subcore_barrier()

    def body(i, carry):
        pltpu.sync_copy(ones_v, acc.at[dst_v.at[i]], add=True)
        return carry

    lax.fori_loop(0, NCHUNK, body, 0)
    plsc.subcore_barrier()

    @pl.when(s < N_IO_TILES)
    def _out():
        sl = pl.ds(s * ROWS_PT, ROWS_PT)
        pltpu.sync_copy(acc.at[sl], out_hbm.at[c, sl])


def _make_sc_agg(D):
    @functools.partial(
        pl.kernel,
        out_type=jax.ShapeDtypeStruct((NC, N, D), jnp.float32),
        mesh=_MESH,
        scratch_types=[
            pltpu.VMEM((NCHUNK, CHUNK), jnp.int32),
            pltpu.VMEM((NCHUNK, CHUNK), jnp.int32),
            pltpu.VMEM((CHUNK, D), jnp.float32),
            pltpu.VMEM((CHUNK, D), jnp.float32),
            pltpu.VMEM_SHARED((NROW, D), jnp.float32),
            pltpu.SemaphoreType.DMA,
            pltpu.SemaphoreType.DMA,
        ],
        compiler_params=_SC_PARAMS,
    )
    def _sc_agg(hp_hbm, src3_hbm, dst3_hbm, zeros_hbm, out_hbm,
                src_v, dst_v, rows0, rows1, acc, sem0, sem1):
        c = lax.axis_index("c")
        s = lax.axis_index("s")
        wid = _wid()

        @pl.when(s < N_IO_TILES)
        def _zero():
            sl = pl.ds(s * ROWS_PT, ROWS_PT)
            pltpu.sync_copy(zeros_hbm.at[sl], acc.at[sl])

        pltpu.sync_copy(src3_hbm.at[wid], src_v)
        pltpu.sync_copy(dst3_hbm.at[wid], dst_v)
        plsc.subcore_barrier()

        def _start(i, buf, sem):
            pltpu.async_copy(hp_hbm.at[src_v.at[i]], buf, sem)

        def _finish(i, buf, sem):
            pltpu.make_async_copy(hp_hbm.at[src_v.at[i]], buf, sem).wait()
            pltpu.sync_copy(buf, acc.at[dst_v.at[i]], add=True)

        _start(0, rows0, sem0)

        def body(j, carry):
            i0 = 2 * j
            _start(i0 + 1, rows1, sem1)
            _finish(i0, rows0, sem0)

            @pl.when(i0 + 2 < NCHUNK)
            def _():
                _start(i0 + 2, rows0, sem0)

            _finish(i0 + 1, rows1, sem1)
            return carry

        lax.fori_loop(0, NCHUNK // 2, body, 0)
        plsc.subcore_barrier()

        @pl.when(s < N_IO_TILES)
        def _out():
            sl = pl.ds(s * ROWS_PT, ROWS_PT)
            pltpu.sync_copy(acc.at[sl], out_hbm.at[c, sl])

    return _sc_agg


_sc_agg_128 = _make_sc_agg(D_HID)
_sc_agg_16 = _make_sc_agg(D_PAD)


_BR = 1000
_GRID = N // _BR


def _tc1_body(x_ref, w1_ref, d0_ref, d1_ref, hp_ref, dis_ref):
    deg = d0_ref[...] + d1_ref[...] + 1.0
    dis = lax.rsqrt(deg)
    h = jnp.dot(x_ref[...], w1_ref[...], preferred_element_type=jnp.float32,
                precision=lax.Precision.HIGHEST)
    hp_ref[...] = h * dis
    dis_ref[...] = dis


def _tc1(x, W1, d0, d1):
    return pl.pallas_call(
        _tc1_body,
        grid=(_GRID,),
        in_specs=[
            pl.BlockSpec((_BR, D_IN), lambda i: (i, 0)),
            pl.BlockSpec((D_IN, D_HID), lambda i: (0, 0)),
            pl.BlockSpec((_BR, 1), lambda i: (i, 0)),
            pl.BlockSpec((_BR, 1), lambda i: (i, 0)),
        ],
        out_specs=[
            pl.BlockSpec((_BR, D_HID), lambda i: (i, 0)),
            pl.BlockSpec((_BR, 1), lambda i: (i, 0)),
        ],
        out_shape=[
            jax.ShapeDtypeStruct((N, D_HID), jnp.float32),
            jax.ShapeDtypeStruct((N, 1), jnp.float32),
        ],
    )(x, W1, d0, d1)


def _tc2_body(p0_ref, p1_ref, hp_ref, dis_ref, b1_ref, w2_ref, h2p_ref):
    dis = dis_ref[...]
    z = (p0_ref[...] + p1_ref[...] + hp_ref[...]) * dis + b1_ref[...]
    z = jnp.maximum(z, 0.0)
    h2 = jnp.dot(z, w2_ref[...], preferred_element_type=jnp.float32,
                 precision=lax.Precision.HIGHEST)
    h2p_ref[...] = h2 * dis


def _tc2(p0, p1, hp, dis, b1, W2p):
    return pl.pallas_call(
        _tc2_body,
        grid=(_GRID,),
        in_specs=[
            pl.BlockSpec((_BR, D_HID), lambda i: (i, 0)),
            pl.BlockSpec((_BR, D_HID), lambda i: (i, 0)),
            pl.BlockSpec((_BR, D_HID), lambda i: (i, 0)),
            pl.BlockSpec((_BR, 1), lambda i: (i, 0)),
            pl.BlockSpec((1, D_HID), lambda i: (0, 0)),
            pl.BlockSpec((D_HID, D_PAD), lambda i: (0, 0)),
        ],
        out_specs=pl.BlockSpec((_BR, D_PAD), lambda i: (i, 0)),
        out_shape=jax.ShapeDtypeStruct((N, D_PAD), jnp.float32),
    )(p0, p1, hp, dis, b1, W2p)


def _tc3_body(q0_ref, q1_ref, h2p_ref, dis_ref, b2_ref, out_ref):
    out_ref[...] = (q0_ref[...] + q1_ref[...] + h2p_ref[...]) * dis_ref[...] \
        + b2_ref[...]


def _tc3(q0, q1, h2p, dis, b2p):
    return pl.pallas_call(
        _tc3_body,
        grid=(_GRID,),
        in_specs=[
            pl.BlockSpec((_BR, D_PAD), lambda i: (i, 0)),
            pl.BlockSpec((_BR, D_PAD), lambda i: (i, 0)),
            pl.BlockSpec((_BR, D_PAD), lambda i: (i, 0)),
            pl.BlockSpec((_BR, 1), lambda i: (i, 0)),
            pl.BlockSpec((1, D_PAD), lambda i: (0, 0)),
        ],
        out_specs=pl.BlockSpec((_BR, D_PAD), lambda i: (i, 0)),
        out_shape=jax.ShapeDtypeStruct((N, D_PAD), jnp.float32),
    )(q0, q1, h2p, dis, b2p)


def kernel(x, edge_index, W1, b1, W2, b2):
    pad = EPAD - E
    src3 = jnp.concatenate(
        [edge_index[0], jnp.zeros((pad,), jnp.int32)]).reshape(NW, NCHUNK, CHUNK)
    dst3 = jnp.concatenate(
        [edge_index[1], jnp.full((pad,), N, jnp.int32)]).reshape(NW, NCHUNK, CHUNK)
    ones16 = jnp.ones((CHUNK, D_PAD), jnp.float32)
    zeros16 = jnp.zeros((N, D_PAD), jnp.float32)
    zeros128 = jnp.zeros((N, D_HID), jnp.float32)
    W2p = jnp.pad(W2, ((0, 0), (0, D_PAD - D_OUT)))
    b1r = b1.reshape(1, D_HID)
    b2p = jnp.pad(b2, (0, D_PAD - D_OUT)).reshape(1, D_PAD)

    degp = _sc_degree(dst3, ones16, zeros16)
    d0 = degp[0, :, 0].reshape(N, 1)
    d1 = degp[1, :, 0].reshape(N, 1)

    hp, dis = _tc1(x, W1, d0, d1)

    aggp = _sc_agg_128(hp, src3, dst3, zeros128)
    h2p = _tc2(aggp[0], aggp[1], hp, dis, b1r, W2p)

    agg2p = _sc_agg_16(h2p, src3, dst3, zeros16)
    out16 = _tc3(agg2p[0], agg2p[1], h2p, dis, b2p)
    return out16[:, :D_OUT]


# speedup vs baseline: 2.0582x; 2.0582x over previous
import functools

import jax
import jax.numpy as jnp
from jax import lax
from jax.experimental import pallas as pl
from jax.experimental.pallas import tpu as pltpu
from jax.experimental.pallas import tpu_sc as plsc

N = 10000
E = 320000
D_IN = 128
D_HID = 128
D_OUT = 2
D_PAD = 16

NC = 2
NS = 16
NW = NC * NS
CHUNK = 80
NCHUNK = 125
EPT = CHUNK * NCHUNK
ROWS_PT = 1000
N_IO_TILES = N // ROWS_PT

_MESH = plsc.VectorSubcoreMesh(
    core_axis_name="c", subcore_axis_name="s", num_cores=NC, num_subcores=NS
)
_SC_PARAMS = pltpu.CompilerParams(use_tc_tiling_on_sc=False)


def _wid():
    return lax.axis_index("c") * NS + lax.axis_index("s")


@functools.partial(
    pl.kernel,
    out_type=jax.ShapeDtypeStruct((NC, N, D_PAD), jnp.float32),
    mesh=_MESH,
    scratch_types=[
        pltpu.VMEM((NCHUNK, CHUNK), jnp.int32),
        pltpu.VMEM((CHUNK, D_PAD), jnp.float32),
        pltpu.VMEM_SHARED((N, D_PAD), jnp.float32),
    ],
    compiler_params=_SC_PARAMS,
)
def _sc_degree(e4_hbm, ones_hbm, zeros_hbm, out_hbm, dst_v, ones_v, acc):
    c = lax.axis_index("c")
    s = lax.axis_index("s")
    wid = _wid()

    @pl.when(s < N_IO_TILES)
    def _zero():
        sl = pl.ds(s * ROWS_PT, ROWS_PT)
        pltpu.sync_copy(zeros_hbm.at[sl], acc.at[sl])

    pltpu.sync_copy(e4_hbm.at[1, wid], dst_v)
    pltpu.sync_copy(ones_hbm, ones_v)
    plsc.subcore_barrier()

    def body(i, carry):
        pltpu.sync_copy(ones_v, acc.at[dst_v.at[i]], add=True)
        return carry

    lax.fori_loop(0, NCHUNK, body, 0)
    plsc.subcore_barrier()

    @pl.when(s < N_IO_TILES)
    def _out():
        sl = pl.ds(s * ROWS_PT, ROWS_PT)
        pltpu.sync_copy(acc.at[sl], out_hbm.at[c, sl])


@functools.partial(
    pl.kernel,
    out_type=jax.ShapeDtypeStruct((NC, N, D_HID), jnp.float32),
    mesh=_MESH,
    scratch_types=[
        pltpu.VMEM((NCHUNK, CHUNK), jnp.int32),
        pltpu.VMEM((NCHUNK, CHUNK), jnp.int32),
        pltpu.VMEM((CHUNK, D_HID), jnp.float32),
        pltpu.VMEM((CHUNK, D_HID), jnp.float32),
        pltpu.VMEM_SHARED((N, D_HID), jnp.float32),
        pltpu.SemaphoreType.DMA,
        pltpu.SemaphoreType.DMA,
    ],
    compiler_params=_SC_PARAMS,
)
def _sc_agg_128(hp_hbm, e4_hbm, zeros_hbm, out_hbm,
                src_v, dst_v, rows0, rows1, acc, sem0, sem1):
    c = lax.axis_index("c")
    s = lax.axis_index("s")
    wid = _wid()

    @pl.when(s < N_IO_TILES)
    def _zero():
        sl = pl.ds(s * ROWS_PT, ROWS_PT)
        pltpu.sync_copy(zeros_hbm.at[sl], acc.at[sl])

    pltpu.sync_copy(e4_hbm.at[0, wid], src_v)
    pltpu.sync_copy(e4_hbm.at[1, wid], dst_v)
    plsc.subcore_barrier()

    def _start(i, buf, sem):
        pltpu.async_copy(hp_hbm.at[src_v.at[i]], buf, sem)

    def _finish(i, buf, sem):
        pltpu.make_async_copy(hp_hbm.at[src_v.at[i]], buf, sem).wait()
        pltpu.sync_copy(buf, acc.at[dst_v.at[i]], add=True)

    _start(0, rows0, sem0)

    def body(j, carry):
        i0 = 2 * j
        _start(i0 + 1, rows1, sem1)
        _finish(i0, rows0, sem0)
        _start(i0 + 2, rows0, sem0)
        _finish(i0 + 1, rows1, sem1)
        return carry

    lax.fori_loop(0, (NCHUNK - 1) // 2, body, 0)
    _finish(NCHUNK - 1, rows0, sem0)
    plsc.subcore_barrier()

    @pl.when(s < N_IO_TILES)
    def _out():
        sl = pl.ds(s * ROWS_PT, ROWS_PT)
        pltpu.sync_copy(acc.at[sl], out_hbm.at[c, sl])


@functools.partial(
    pl.kernel,
    out_type=jax.ShapeDtypeStruct((NC, N, D_PAD), jnp.float32),
    mesh=_MESH,
    scratch_types=[
        pltpu.VMEM((NCHUNK, CHUNK), jnp.int32),
        pltpu.VMEM((NCHUNK, CHUNK), jnp.int32),
        pltpu.VMEM((CHUNK, D_PAD), jnp.float32),
        pltpu.VMEM((CHUNK, D_PAD), jnp.float32),
        pltpu.VMEM_SHARED((N, D_PAD), jnp.float32),
        pltpu.VMEM_SHARED((N, D_PAD), jnp.float32),
        pltpu.SemaphoreType.DMA,
        pltpu.SemaphoreType.DMA,
    ],
    compiler_params=_SC_PARAMS,
)
def _sc_agg_16(h2p_hbm, e4_hbm, zeros_hbm, out_hbm,
               src_v, dst_v, rows0, rows1, acc, table, sem0, sem1):
    c = lax.axis_index("c")
    s = lax.axis_index("s")
    wid = _wid()

    @pl.when(s < N_IO_TILES)
    def _zero():
        sl = pl.ds(s * ROWS_PT, ROWS_PT)
        pltpu.sync_copy(zeros_hbm.at[sl], acc.at[sl])
        pltpu.sync_copy(h2p_hbm.at[sl], table.at[sl])

    pltpu.sync_copy(e4_hbm.at[0, wid], src_v)
    pltpu.sync_copy(e4_hbm.at[1, wid], dst_v)
    plsc.subcore_barrier()

    def _start(i, buf, sem):
        pltpu.async_copy(table.at[src_v.at[i]], buf, sem)

    def _finish(i, buf, sem):
        pltpu.make_async_copy(table.at[src_v.at[i]], buf, sem).wait()
        pltpu.sync_copy(buf, acc.at[dst_v.at[i]], add=True)

    _start(0, rows0, sem0)

    def body(j, carry):
        i0 = 2 * j
        _start(i0 + 1, rows1, sem1)
        _finish(i0, rows0, sem0)
        _start(i0 + 2, rows0, sem0)
        _finish(i0 + 1, rows1, sem1)
        return carry

    lax.fori_loop(0, (NCHUNK - 1) // 2, body, 0)
    _finish(NCHUNK - 1, rows0, sem0)
    plsc.subcore_barrier()

    @pl.when(s < N_IO_TILES)
    def _out():
        sl = pl.ds(s * ROWS_PT, ROWS_PT)
        pltpu.sync_copy(acc.at[sl], out_hbm.at[c, sl])


_BR = 2000
_GRID = N // _BR


def _tc1a_body(x_ref, w1_ref, h_ref):
    h_ref[...] = jnp.dot(x_ref[...], w1_ref[...],
                         preferred_element_type=jnp.float32,
                         precision=lax.Precision.HIGHEST)


def _tc1a(x, W1):
    return pl.pallas_call(
        _tc1a_body,
        grid=(_GRID,),
        in_specs=[
            pl.BlockSpec((_BR, D_IN), lambda i: (i, 0)),
            pl.BlockSpec((D_IN, D_HID), lambda i: (0, 0)),
        ],
        out_specs=pl.BlockSpec((_BR, D_HID), lambda i: (i, 0)),
        out_shape=jax.ShapeDtypeStruct((N, D_HID), jnp.float32),
    )(x, W1)


def _tc1b_body(h_ref, dp0_ref, dp1_ref, hp_ref, dis_ref):
    deg = dp0_ref[0, :, 0:1] + dp1_ref[0, :, 0:1] + 1.0
    dis = lax.rsqrt(deg)
    hp_ref[...] = h_ref[...] * dis
    dis_ref[...] = dis


def _tc1b(h, degp):
    return pl.pallas_call(
        _tc1b_body,
        grid=(_GRID,),
        in_specs=[
            pl.BlockSpec((_BR, D_HID), lambda i: (i, 0)),
            pl.BlockSpec((1, _BR, D_PAD), lambda i: (0, i, 0)),
            pl.BlockSpec((1, _BR, D_PAD), lambda i: (1, i, 0)),
        ],
        out_specs=[
            pl.BlockSpec((_BR, D_HID), lambda i: (i, 0)),
            pl.BlockSpec((_BR, 1), lambda i: (i, 0)),
        ],
        out_shape=[
            jax.ShapeDtypeStruct((N, D_HID), jnp.float32),
            jax.ShapeDtypeStruct((N, 1), jnp.float32),
        ],
    )(h, degp, degp)


def _tc2_body(p_ref, q_ref, hp_ref, dis_ref, b1_ref, w2_ref, h2p_ref):
    dis = dis_ref[...]
    z = (p_ref[0] + q_ref[0] + hp_ref[...]) * dis + b1_ref[...]
    z = jnp.maximum(z, 0.0)
    h2 = jnp.dot(z, w2_ref[...], preferred_element_type=jnp.float32,
                 precision=lax.Precision.HIGHEST)
    h2p_ref[...] = h2 * dis


def _tc2(aggp, hp, dis, b1r, W2p):
    return pl.pallas_call(
        _tc2_body,
        grid=(_GRID,),
        in_specs=[
            pl.BlockSpec((1, _BR, D_HID), lambda i: (0, i, 0)),
            pl.BlockSpec((1, _BR, D_HID), lambda i: (1, i, 0)),
            pl.BlockSpec((_BR, D_HID), lambda i: (i, 0)),
            pl.BlockSpec((_BR, 1), lambda i: (i, 0)),
            pl.BlockSpec((1, D_HID), lambda i: (0, 0)),
            pl.BlockSpec((D_HID, D_PAD), lambda i: (0, 0)),
        ],
        out_specs=pl.BlockSpec((_BR, D_PAD), lambda i: (i, 0)),
        out_shape=jax.ShapeDtypeStruct((N, D_PAD), jnp.float32),
    )(aggp, aggp, hp, dis, b1r, W2p)


def _tc3_body(p_ref, q_ref, h2p_ref, dis_ref, b2_ref, out_ref):
    full = (p_ref[0] + q_ref[0] + h2p_ref[...]) * dis_ref[...] + b2_ref[...]
    out_ref[...] = full[:, :D_OUT]


def _tc3(agg2p, h2p, dis, b2p):
    return pl.pallas_call(
        _tc3_body,
        grid=(_GRID,),
        in_specs=[
            pl.BlockSpec((1, _BR, D_PAD), lambda i: (0, i, 0)),
            pl.BlockSpec((1, _BR, D_PAD), lambda i: (1, i, 0)),
            pl.BlockSpec((_BR, D_PAD), lambda i: (i, 0)),
            pl.BlockSpec((_BR, 1), lambda i: (i, 0)),
            pl.BlockSpec((1, D_PAD), lambda i: (0, 0)),
        ],
        out_specs=pl.BlockSpec((_BR, D_OUT), lambda i: (i, 0)),
        out_shape=jax.ShapeDtypeStruct((N, D_OUT), jnp.float32),
    )(agg2p, agg2p, h2p, dis, b2p)


def kernel(x, edge_index, W1, b1, W2, b2):
    e4 = edge_index.reshape(2, NW, NCHUNK, CHUNK)
    ones16 = jnp.ones((CHUNK, D_PAD), jnp.float32)
    zeros16 = jnp.zeros((N, D_PAD), jnp.float32)
    zeros128 = jnp.zeros((N, D_HID), jnp.float32)
    W2p = jnp.pad(W2, ((0, 0), (0, D_PAD - D_OUT)))
    b1r = b1.reshape(1, D_HID)
    b2p = jnp.pad(b2, (0, D_PAD - D_OUT)).reshape(1, D_PAD)

    h = _tc1a(x, W1)
    degp = _sc_degree(e4, ones16, zeros16)
    hp, dis = _tc1b(h, degp)

    aggp = _sc_agg_128(hp, e4, zeros128)
    h2p = _tc2(aggp, hp, dis, b1r, W2p)

    agg2p = _sc_agg_16(h2p, e4, zeros16)
    return _tc3(agg2p, h2p, dis, b2p)
